# baseline (speedup 1.0000x reference)
import functools

import jax
import jax.numpy as jnp
from jax import lax
from jax.experimental import pallas as pl
from jax.experimental.pallas import tpu as pltpu
from jax.experimental.pallas import tpu_sc as plsc

_B = 2
_C = 8
_NPIX = 1048576
_NPART = 2097152
_R = _B * _C

_NC = 2
_NS = 16
_NW = _NC * _NS
_PER_W = _NPART // _NW
_CHUNK = 2048
_NCHUNK = _PER_W // _CHUNK

_mesh = plsc.VectorSubcoreMesh(
    core_axis_name="c", subcore_axis_name="s", num_cores=_NC, num_subcores=_NS
)


@functools.partial(
    pl.kernel,
    out_type=jax.ShapeDtypeStruct((_NPART, _R), jnp.float32),
    mesh=_mesh,
    scratch_types=[
        pltpu.VMEM((_CHUNK,), jnp.int32),
        pltpu.VMEM((_CHUNK,), jnp.int32),
        pltpu.VMEM((_CHUNK, _R), jnp.float32),
        pltpu.VMEM((_CHUNK, _R), jnp.float32),
        pltpu.SemaphoreType.DMA,
        pltpu.SemaphoreType.DMA,
    ],
    compiler_params=pltpu.CompilerParams(
        use_tc_tiling_on_sc=False,
        needs_layout_passes=False,
        disable_bounds_checks=True,
    ),
)
def _sc_gather(table_hbm, idx_hbm, out_hbm, idx_a, idx_b, rows_a, rows_b,
               sem_a, sem_b):
    wid = lax.axis_index("s") * _NC + lax.axis_index("c")
    base = wid * _PER_W

    def fetch(k, idx_v, rows_v, sem):
        pltpu.sync_copy(idx_hbm.at[pl.ds(base + k * _CHUNK, _CHUNK)], idx_v)
        pltpu.async_copy(table_hbm.at[idx_v], rows_v, sem)

    def gather_wait(idx_v, rows_v, sem):
        pltpu.make_async_copy(table_hbm.at[idx_v], rows_v, sem).wait()

    def store(rows_v, off):
        pltpu.sync_copy(rows_v, out_hbm.at[pl.ds(off, _CHUNK), :])

    fetch(0, idx_a, rows_a, sem_a)

    def body(i, carry):
        k = 2 * i
        fetch(k + 1, idx_b, rows_b, sem_b)
        gather_wait(idx_a, rows_a, sem_a)
        store(rows_a, base + k * _CHUNK)

        @pl.when(i < _NCHUNK // 2 - 1)
        def _():
            fetch(k + 2, idx_a, rows_a, sem_a)

        gather_wait(idx_b, rows_b, sem_b)
        store(rows_b, base + (k + 1) * _CHUNK)
        return carry

    lax.fori_loop(0, _NCHUNK // 2, body, 0)


_BM = 512
_BP = _BM * 8


def _tc_transpose_body(x_ref, o_ref):
    x = x_ref[...]
    r_iota = lax.broadcasted_iota(jnp.int32, (_R, 128), 0)
    q_iota = lax.broadcasted_iota(jnp.int32, (_R, 128), 1)
    for j in range(8):
        ej = (q_iota == _R * j + r_iota).astype(jnp.float32)
        yj = lax.dot_general(
            ej, x, (((1,), (1,)), ((), ())),
            precision=lax.Precision.HIGHEST,
            preferred_element_type=jnp.float32)
        o_ref[:, :, j * _BM:(j + 1) * _BM] = yj.reshape(_B, _C, _BM)


_tc_transpose = pl.pallas_call(
    _tc_transpose_body,
    grid=(_NPART // _BP,),
    in_specs=[pl.BlockSpec((_BM, 128), lambda i: (i, 0))],
    out_specs=pl.BlockSpec((_B, _C, _BP), lambda i: (0, 0, i)),
    out_shape=jax.ShapeDtypeStruct((_B, _C, _NPART), jnp.float32),
)


def kernel(input_features, aprs, level_deltas):
    del level_deltas
    table = input_features.reshape(_R, _NPIX).T
    idxp = aprs.reshape(-1, 8, _BM).transpose(0, 2, 1).reshape(-1)
    rows = _sc_gather(table, idxp)
    return _tc_transpose(rows.reshape(_NPART // 8, 128))


# speedup vs baseline: 1.6084x; 1.6084x over previous
import functools

import jax
import jax.numpy as jnp
from jax import lax
from jax.experimental import pallas as pl
from jax.experimental.pallas import tpu as pltpu
from jax.experimental.pallas import tpu_sc as plsc

_B = 2
_C = 8
_NPIX = 1048576
_NPART = 2097152
_R = _B * _C

_NC = 2
_NS = 16
_NW = _NC * _NS
_PER_W = _NPART // _NW
_CHUNK = 2048
_NCHUNK = _PER_W // _CHUNK

_mesh = plsc.VectorSubcoreMesh(
    core_axis_name="c", subcore_axis_name="s", num_cores=_NC, num_subcores=_NS
)


@functools.partial(
    pl.kernel,
    out_type=jax.ShapeDtypeStruct((_NPART, _R), jnp.float32),
    mesh=_mesh,
    scratch_types=[
        pltpu.VMEM((_CHUNK,), jnp.int32),
        pltpu.VMEM((_CHUNK,), jnp.int32),
        pltpu.VMEM((_CHUNK, _R), jnp.float32),
        pltpu.VMEM((_CHUNK, _R), jnp.float32),
        pltpu.SemaphoreType.DMA,
        pltpu.SemaphoreType.DMA,
    ],
    compiler_params=pltpu.CompilerParams(
        use_tc_tiling_on_sc=False,
        needs_layout_passes=False,
        disable_bounds_checks=True,
    ),
)
def _sc_gather(table_hbm, idx_hbm, out_hbm, idx_a, idx_b, rows_a, rows_b,
               sem_a, sem_b):
    wid = lax.axis_index("s") * _NC + lax.axis_index("c")
    base = wid * _PER_W

    def fetch(k, idx_v, rows_v, sem):
        pltpu.sync_copy(idx_hbm.at[pl.ds(base + k * _CHUNK, _CHUNK)], idx_v)
        pltpu.async_copy(table_hbm.at[idx_v], rows_v, sem)

    def gather_wait(idx_v, rows_v, sem):
        pltpu.make_async_copy(table_hbm.at[idx_v], rows_v, sem).wait()

    def store(rows_v, off):
        pltpu.sync_copy(rows_v, out_hbm.at[pl.ds(off, _CHUNK), :])

    fetch(0, idx_a, rows_a, sem_a)

    def body(i, carry):
        k = 2 * i
        fetch(k + 1, idx_b, rows_b, sem_b)
        gather_wait(idx_a, rows_a, sem_a)
        store(rows_a, base + k * _CHUNK)

        @pl.when(i < _NCHUNK // 2 - 1)
        def _():
            fetch(k + 2, idx_a, rows_a, sem_a)

        gather_wait(idx_b, rows_b, sem_b)
        store(rows_b, base + (k + 1) * _CHUNK)
        return carry

    lax.fori_loop(0, _NCHUNK // 2, body, 0)


_BM = 512
_BP = _BM * 8


def _tc_transpose_body(x_ref, o_ref):
    x = x_ref[...]
    y = x.T
    for j in range(8):
        o_ref[:, :, j * _BM:(j + 1) * _BM] = (
            y[_R * j:_R * (j + 1), :].reshape(_B, _C, _BM))


_tc_transpose = pl.pallas_call(
    _tc_transpose_body,
    grid=(_NPART // _BP,),
    in_specs=[pl.BlockSpec((_BM, 128), lambda i: (i, 0))],
    out_specs=pl.BlockSpec((_B, _C, _BP), lambda i: (0, 0, i)),
    out_shape=jax.ShapeDtypeStruct((_B, _C, _NPART), jnp.float32),
)


def kernel(input_features, aprs, level_deltas):
    del level_deltas
    table = input_features.reshape(_R, _NPIX).T
    idxp = aprs.reshape(-1, 8, _BM).transpose(0, 2, 1).reshape(-1)
    rows = _sc_gather(table, idxp)
    return _tc_transpose(rows.reshape(_NPART // 8, 128))


# speedup vs baseline: 2.0071x; 1.2479x over previous
import functools

import jax
import jax.numpy as jnp
from jax import lax
from jax.experimental import pallas as pl
from jax.experimental.pallas import tpu as pltpu
from jax.experimental.pallas import tpu_sc as plsc

_B = 2
_C = 8
_NPIX = 1048576
_NPART = 2097152
_R = _B * _C

_NC = 2
_NS = 16
_NW = _NC * _NS
_PER_W = _NPART // _NW
_CHUNK = 2048
_NCHUNK = _PER_W // _CHUNK

_mesh = plsc.VectorSubcoreMesh(
    core_axis_name="c", subcore_axis_name="s", num_cores=_NC, num_subcores=_NS
)


@functools.partial(
    pl.kernel,
    out_type=jax.ShapeDtypeStruct((_NPART, _R), jnp.float32),
    mesh=_mesh,
    scratch_types=[
        pltpu.VMEM((_CHUNK,), jnp.int32),
        pltpu.VMEM((_CHUNK,), jnp.int32),
        pltpu.VMEM((_CHUNK, _R), jnp.float32),
        pltpu.VMEM((_CHUNK, _R), jnp.float32),
        pltpu.SemaphoreType.DMA,
        pltpu.SemaphoreType.DMA,
    ],
    compiler_params=pltpu.CompilerParams(
        use_tc_tiling_on_sc=False,
        needs_layout_passes=False,
        disable_bounds_checks=True,
    ),
)
def _sc_gather(table_hbm, idx_hbm, out_hbm, idx_a, idx_b, rows_a, rows_b,
               sem_a, sem_b):
    wid = lax.axis_index("s") * _NC + lax.axis_index("c")
    base = wid * _PER_W

    def fetch(k, idx_v, rows_v, sem):
        pltpu.sync_copy(idx_hbm.at[pl.ds(base + k * _CHUNK, _CHUNK)], idx_v)
        pltpu.async_copy(table_hbm.at[idx_v], rows_v, sem)

    def gather_wait(idx_v, rows_v, sem):
        pltpu.make_async_copy(table_hbm.at[idx_v], rows_v, sem).wait()

    def store(rows_v, off):
        pltpu.sync_copy(rows_v, out_hbm.at[pl.ds(off, _CHUNK), :])

    fetch(0, idx_a, rows_a, sem_a)

    def body(i, carry):
        k = 2 * i
        fetch(k + 1, idx_b, rows_b, sem_b)
        gather_wait(idx_a, rows_a, sem_a)
        store(rows_a, base + k * _CHUNK)

        @pl.when(i < _NCHUNK // 2 - 1)
        def _():
            fetch(k + 2, idx_a, rows_a, sem_a)

        gather_wait(idx_b, rows_b, sem_b)
        store(rows_b, base + (k + 1) * _CHUNK)
        return carry

    lax.fori_loop(0, _NCHUNK // 2, body, 0)


_BM = 2048
_BP = _BM * 8


def _tc_transpose_body(x_ref, o_ref):
    x = x_ref[...]
    y = x.T
    for j in range(8):
        o_ref[:, :, j * _BM:(j + 1) * _BM] = (
            y[_R * j:_R * (j + 1), :].reshape(_B, _C, _BM))


_tc_transpose = pl.pallas_call(
    _tc_transpose_body,
    grid=(_NPART // _BP,),
    in_specs=[pl.BlockSpec((_BM, 128), lambda i: (i, 0))],
    out_specs=pl.BlockSpec((_B, _C, _BP), lambda i: (0, 0, i)),
    out_shape=jax.ShapeDtypeStruct((_B, _C, _NPART), jnp.float32),
)


def kernel(input_features, aprs, level_deltas):
    del level_deltas
    table = input_features.reshape(_R, _NPIX).T
    idxp = aprs.reshape(-1, 8, _BM).transpose(0, 2, 1).reshape(-1)
    rows = _sc_gather(table, idxp)
    return _tc_transpose(rows.reshape(_NPART // 8, 128))


# speedup vs baseline: 2.1750x; 1.0837x over previous
import functools

import jax
import jax.numpy as jnp
from jax import lax
from jax.experimental import pallas as pl
from jax.experimental.pallas import tpu as pltpu
from jax.experimental.pallas import tpu_sc as plsc

_B = 2
_C = 8
_NPIX = 1048576
_NPART = 2097152
_R = _B * _C

_NC = 2
_NS = 16
_NW = _NC * _NS
_PER_W = _NPART // _NW
_CHUNK = 2048
_NCHUNK = _PER_W // _CHUNK

_mesh = plsc.VectorSubcoreMesh(
    core_axis_name="c", subcore_axis_name="s", num_cores=_NC, num_subcores=_NS
)


@functools.partial(
    pl.kernel,
    out_type=jax.ShapeDtypeStruct((_NPART, _R), jnp.float32),
    mesh=_mesh,
    scratch_types=[
        pltpu.VMEM((_CHUNK,), jnp.int32),
        pltpu.VMEM((_CHUNK,), jnp.int32),
        pltpu.VMEM((_CHUNK, _R), jnp.float32),
        pltpu.VMEM((_CHUNK, _R), jnp.float32),
        pltpu.SemaphoreType.DMA,
        pltpu.SemaphoreType.DMA,
    ],
    compiler_params=pltpu.CompilerParams(
        use_tc_tiling_on_sc=False,
        needs_layout_passes=False,
        disable_bounds_checks=True,
    ),
)
def _sc_gather(table_hbm, idx_hbm, out_hbm, idx_a, idx_b, rows_a, rows_b,
               sem_a, sem_b):
    wid = lax.axis_index("s") * _NC + lax.axis_index("c")
    base = wid * _PER_W

    def fetch(k, idx_v, rows_v, sem):
        pltpu.sync_copy(idx_hbm.at[pl.ds(base + k * _CHUNK, _CHUNK)], idx_v)
        pltpu.async_copy(table_hbm.at[idx_v], rows_v, sem)

    def gather_wait(idx_v, rows_v, sem):
        pltpu.make_async_copy(table_hbm.at[idx_v], rows_v, sem).wait()

    def store(rows_v, off):
        pltpu.sync_copy(rows_v, out_hbm.at[pl.ds(off, _CHUNK), :])

    fetch(0, idx_a, rows_a, sem_a)

    def body(i, carry):
        k = 2 * i
        fetch(k + 1, idx_b, rows_b, sem_b)
        gather_wait(idx_a, rows_a, sem_a)
        store(rows_a, base + k * _CHUNK)

        @pl.when(i < _NCHUNK // 2 - 1)
        def _():
            fetch(k + 2, idx_a, rows_a, sem_a)

        gather_wait(idx_b, rows_b, sem_b)
        store(rows_b, base + (k + 1) * _CHUNK)
        return carry

    lax.fori_loop(0, _NCHUNK // 2, body, 0)


_BM = 8192
_BP = _BM * 8


def _tc_transpose_body(x_ref, o_ref):
    x = x_ref[...]
    y = x.T
    for j in range(8):
        o_ref[:, :, j * _BM:(j + 1) * _BM] = (
            y[_R * j:_R * (j + 1), :].reshape(_B, _C, _BM))


_tc_transpose = pl.pallas_call(
    _tc_transpose_body,
    grid=(_NPART // _BP,),
    in_specs=[pl.BlockSpec((_BM, 128), lambda i: (i, 0))],
    out_specs=pl.BlockSpec((_B, _C, _BP), lambda i: (0, 0, i)),
    out_shape=jax.ShapeDtypeStruct((_B, _C, _NPART), jnp.float32),
)


def kernel(input_features, aprs, level_deltas):
    del level_deltas
    table = input_features.reshape(_R, _NPIX).T
    idxp = aprs.reshape(-1, 8, _BM).transpose(0, 2, 1).reshape(-1)
    rows = _sc_gather(table, idxp)
    return _tc_transpose(rows.reshape(_NPART // 8, 128))


# speedup vs baseline: 3.4722x; 1.5964x over previous
import functools

import jax
import jax.numpy as jnp
from jax import lax
from jax.experimental import pallas as pl
from jax.experimental.pallas import tpu as pltpu
from jax.experimental.pallas import tpu_sc as plsc

_B = 2
_C = 8
_NPIX = 1048576
_NPART = 2097152
_R = _B * _C

_NC = 2
_NS = 16
_NW = _NC * _NS
_PER_W = _NPART // _NW
_CHUNK = 2048
_NCHUNK = _PER_W // _CHUNK

_mesh = plsc.VectorSubcoreMesh(
    core_axis_name="c", subcore_axis_name="s", num_cores=_NC, num_subcores=_NS
)


@functools.partial(
    pl.kernel,
    out_type=jax.ShapeDtypeStruct((_NPART, _R), jnp.float32),
    mesh=_mesh,
    scratch_types=[
        pltpu.VMEM((_CHUNK,), jnp.int32),
        pltpu.VMEM((_CHUNK,), jnp.int32),
        pltpu.VMEM((_CHUNK, _R), jnp.float32),
        pltpu.VMEM((_CHUNK, _R), jnp.float32),
        pltpu.SemaphoreType.DMA,
        pltpu.SemaphoreType.DMA,
    ],
    compiler_params=pltpu.CompilerParams(
        use_tc_tiling_on_sc=False,
        needs_layout_passes=False,
        disable_bounds_checks=True,
    ),
)
def _sc_gather(table_hbm, idx_hbm, out_hbm, idx_a, idx_b, rows_a, rows_b,
               sem_a, sem_b):
    wid = lax.axis_index("s") * _NC + lax.axis_index("c")
    base = wid * _PER_W

    def fetch(k, idx_v, rows_v, sem):
        pltpu.sync_copy(idx_hbm.at[pl.ds(base + k * _CHUNK, _CHUNK)], idx_v)
        pltpu.async_copy(table_hbm.at[idx_v], rows_v, sem)

    def gather_wait(idx_v, rows_v, sem):
        pltpu.make_async_copy(table_hbm.at[idx_v], rows_v, sem).wait()

    def store(rows_v, off):
        pltpu.sync_copy(rows_v, out_hbm.at[pl.ds(off, _CHUNK), :])

    fetch(0, idx_a, rows_a, sem_a)

    def body(i, carry):
        k = 2 * i
        fetch(k + 1, idx_b, rows_b, sem_b)
        gather_wait(idx_a, rows_a, sem_a)
        store(rows_a, base + k * _CHUNK)

        @pl.when(i < _NCHUNK // 2 - 1)
        def _():
            fetch(k + 2, idx_a, rows_a, sem_a)

        gather_wait(idx_b, rows_b, sem_b)
        store(rows_b, base + (k + 1) * _CHUNK)
        return carry

    lax.fori_loop(0, _NCHUNK // 2, body, 0)


_BV = 8192
_BVd8 = _BV // 8


def _tc_table_body(x_ref, o_ref):
    x = x_ref[...]
    y = x.reshape(_R, _BV)
    z = jnp.concatenate(
        [y[:, j * _BVd8:(j + 1) * _BVd8] for j in range(8)], axis=0)
    o_ref[...] = z.T


_tc_table = pl.pallas_call(
    _tc_table_body,
    grid=(_NPIX // _BV,),
    in_specs=[pl.BlockSpec((_B, _C, _BV), lambda i: (0, 0, i))],
    out_specs=pl.BlockSpec((_BVd8, 128), lambda i: (i, 0)),
    out_shape=jax.ShapeDtypeStruct((_NPIX // 8, 128), jnp.float32),
)


_BM = 8192
_BP = _BM * 8


def _tc_transpose_body(x_ref, o_ref):
    x = x_ref[...]
    y = x.T
    for j in range(8):
        o_ref[:, :, j * _BM:(j + 1) * _BM] = (
            y[_R * j:_R * (j + 1), :].reshape(_B, _C, _BM))


_tc_transpose = pl.pallas_call(
    _tc_transpose_body,
    grid=(_NPART // _BP,),
    in_specs=[pl.BlockSpec((_BM, 128), lambda i: (i, 0))],
    out_specs=pl.BlockSpec((_B, _C, _BP), lambda i: (0, 0, i)),
    out_shape=jax.ShapeDtypeStruct((_B, _C, _NPART), jnp.float32),
)


def kernel(input_features, aprs, level_deltas):
    del level_deltas
    table = _tc_table(input_features).reshape(_NPIX, _R)
    g = ((aprs & ~(_BV - 1)) | ((aprs & (_BVd8 - 1)) << 3)
         | ((aprs >> 10) & 7))
    idxp = g.reshape(-1, 8, _BM).transpose(0, 2, 1).reshape(-1)
    rows = _sc_gather(table, idxp)
    return _tc_transpose(rows.reshape(_NPART // 8, 128))
